# baseline (speedup 1.0000x reference)
import functools

import jax
import jax.numpy as jnp
from jax import lax
from jax.experimental import pallas as pl
from jax.experimental.pallas import tpu as pltpu
from jax.experimental.pallas import tpu_sc as plsc

MASK_VALUE = 119.0
MASK_IDX = 0.0

B, N, D, M = 256, 128, 128, 32
NC, NS = 2, 16
NW = NC * NS
GPW = B // NW
L = 16

ROWS = B * N
MPW = GPW * M
IH, IW = MPW // 128, 128
GPW_ROWS = GPW * N

_mesh = plsc.VectorSubcoreMesh(
    core_axis_name="c", subcore_axis_name="s", num_cores=NC, num_subcores=NS
)


@functools.partial(
    pl.kernel,
    out_type=jax.ShapeDtypeStruct((ROWS, D), jnp.float32),
    mesh=_mesh,
    scratch_types=[
        pltpu.VMEM((GPW, M), jnp.int32),
        pltpu.VMEM((IH, IW), jnp.int32),
        pltpu.VMEM((IH, IW, D), jnp.float32),
        pltpu.SemaphoreType.DMA,
    ],
)
def _mask_kernel(nf_hbm, idx_hbm, out_hbm, idx_raw, idx_row, rows_v, sem):
    wid = lax.axis_index("s") * NC + lax.axis_index("c")
    base = wid * GPW

    pltpu.sync_copy(idx_hbm.at[pl.ds(base, GPW)], idx_raw)

    for g in range(GPW):
        boff = (base + g) * N
        for j in range(M // L):
            r = idx_raw[g, pl.ds(j * L, L)]
            rid = r + boff
            lin = g * M + j * L
            idx_row[lin // IW, pl.ds(lin % IW, L)] = rid

    gets = [
        pltpu.async_copy(nf_hbm.at[idx_row.at[h]], rows_v.at[h], sem)
        for h in range(IH)
    ]

    pltpu.sync_copy(
        nf_hbm.at[pl.ds(base * N, GPW_ROWS)],
        out_hbm.at[pl.ds(base * N, GPW_ROWS)],
    )
    for cp in gets:
        cp.wait()

    lane = lax.iota(jnp.int32, L)
    is0 = lane == 0
    is1 = lane == 1
    for h in range(IH):
        for i in range(IW):
            v = rows_v[h, i, pl.ds(0, L)]
            v = jnp.where(is0, MASK_VALUE, v)
            v = jnp.where(is1, MASK_IDX, v)
            rows_v[h, i, pl.ds(0, L)] = v

    puts = [
        pltpu.async_copy(rows_v.at[h], out_hbm.at[idx_row.at[h]], sem)
        for h in range(IH)
    ]
    for cp in puts:
        cp.wait()


def kernel(node_features, adjacency, nodes_to_mask):
    del adjacency
    flat = node_features.reshape(ROWS, D)
    return _mask_kernel(flat, nodes_to_mask).reshape(B, N, D)


# speedup vs baseline: 14.7330x; 14.7330x over previous
import functools

import jax
import jax.numpy as jnp
from jax import lax
from jax.experimental import pallas as pl
from jax.experimental.pallas import tpu as pltpu
from jax.experimental.pallas import tpu_sc as plsc

MASK_VALUE = 119.0
MASK_IDX = 0.0

B, N, D, M = 256, 128, 128, 32
NC, NS = 2, 16
NW = NC * NS
GPW = B // NW
L = 16

ROWS = B * N
MPW = GPW * M
IH, IW = MPW // 128, 128
GPW_ROWS = GPW * N
NT = 4
TROWS = GPW_ROWS // NT

_mesh = plsc.VectorSubcoreMesh(
    core_axis_name="c", subcore_axis_name="s", num_cores=NC, num_subcores=NS
)


@functools.partial(
    pl.kernel,
    out_type=jax.ShapeDtypeStruct((ROWS, D), jnp.float32),
    mesh=_mesh,
    scratch_types=[
        pltpu.VMEM((GPW, M), jnp.int32),
        pltpu.VMEM((IH, IW), jnp.int32),
        pltpu.VMEM((IH, IW, D), jnp.float32),
        [pltpu.VMEM((TROWS, D), jnp.float32) for _ in range(2)],
        pltpu.SemaphoreType.DMA,
        [pltpu.SemaphoreType.DMA for _ in range(2)],
        [pltpu.SemaphoreType.DMA for _ in range(2)],
    ],
)
def _mask_kernel(nf_hbm, idx_hbm, out_hbm, idx_raw, idx_row, rows_v,
                 bufs, sem, sin, sout):
    wid = lax.axis_index("s") * NC + lax.axis_index("c")
    base = wid * GPW

    pltpu.sync_copy(idx_hbm.at[pl.ds(base, GPW)], idx_raw)

    for g in range(GPW):
        boff = (base + g) * N
        for j in range(M // L):
            r = idx_raw[g, pl.ds(j * L, L)]
            rid = r + boff
            lin = g * M + j * L
            idx_row[lin // IW, pl.ds(lin % IW, L)] = rid

    gets = [
        pltpu.async_copy(nf_hbm.at[idx_row.at[h]], rows_v.at[h], sem)
        for h in range(IH)
    ]

    row0 = base * N
    cin = [None, None]
    cout = [None, None]
    cin[0] = pltpu.async_copy(nf_hbm.at[pl.ds(row0, TROWS)], bufs[0], sin[0])
    for t in range(NT):
        s = t % 2
        if t + 1 < NT:
            nxt = (t + 1) % 2
            if cout[nxt] is not None:
                cout[nxt].wait()
            cin[nxt] = pltpu.async_copy(
                nf_hbm.at[pl.ds(row0 + (t + 1) * TROWS, TROWS)],
                bufs[nxt], sin[nxt])
        cin[s].wait()
        cout[s] = pltpu.async_copy(
            bufs[s], out_hbm.at[pl.ds(row0 + t * TROWS, TROWS)], sout[s])
    cout[0].wait()
    cout[1].wait()
    for cp in gets:
        cp.wait()

    lane = lax.iota(jnp.int32, L)
    is0 = lane == 0
    is1 = lane == 1
    for h in range(IH):
        for i in range(IW):
            v = rows_v[h, i, pl.ds(0, L)]
            v = jnp.where(is0, MASK_VALUE, v)
            v = jnp.where(is1, MASK_IDX, v)
            rows_v[h, i, pl.ds(0, L)] = v

    puts = [
        pltpu.async_copy(rows_v.at[h], out_hbm.at[idx_row.at[h]], sem)
        for h in range(IH)
    ]
    for cp in puts:
        cp.wait()


def kernel(node_features, adjacency, nodes_to_mask):
    del adjacency
    flat = node_features.reshape(ROWS, D)
    return _mask_kernel(flat, nodes_to_mask).reshape(B, N, D)
